# baseline (speedup 1.0000x reference)
import functools

import jax
import jax.numpy as jnp
from jax import lax
from jax.experimental import pallas as pl
from jax.experimental.pallas import tpu as pltpu
from jax.experimental.pallas import tpu_sc as plsc

N = 10000
E = 320000
F = 128

NC = 2
NS = 16
NW = NC * NS
CHUNK = 128
T_CH = 80
E_PAD = NW * CHUNK * T_CH
RPT = 624
RPT_LAST = N - (NS - 1) * RPT
N_ACC = N + 8
DW = F

_mesh = plsc.VectorSubcoreMesh(core_axis_name="c", subcore_axis_name="s")


def _stripe(s, fn):
    @pl.when(s < NS - 1)
    def _():
        fn(pl.multiple_of(s * RPT, 8), RPT)

    @pl.when(s == NS - 1)
    def _():
        fn((NS - 1) * RPT, RPT_LAST)


def _deg_body(dst_hbm, zeros_hbm, ones_hbm, out_hbm, di2d, ones_v, acc):
    c = lax.axis_index("c")
    s = lax.axis_index("s")
    wid = c * NS + s
    _stripe(s, lambda r0, nr: pltpu.sync_copy(
        zeros_hbm.at[pl.ds(r0, nr)], acc.at[pl.ds(r0, nr)]))
    pltpu.sync_copy(ones_hbm, ones_v)
    pltpu.sync_copy(dst_hbm.at[wid], di2d)
    plsc.subcore_barrier()

    def step(i, carry):
        pltpu.sync_copy(ones_v, acc.at[di2d.at[i]], add=True)
        return carry

    lax.fori_loop(0, T_CH, step, 0)
    plsc.subcore_barrier()
    _stripe(s, lambda r0, nr: pltpu.sync_copy(
        acc.at[pl.ds(r0, nr)], out_hbm.at[c, pl.ds(r0, nr)]))


_deg_call = functools.partial(
    pl.kernel,
    out_type=jax.ShapeDtypeStruct((NC, N, DW), jnp.float32),
    mesh=_mesh,
    scratch_types=[
        pltpu.VMEM((T_CH, CHUNK), jnp.int32),
        pltpu.VMEM((CHUNK, DW), jnp.float32),
        pltpu.VMEM_SHARED((N_ACC, DW), jnp.float32),
    ],
)(_deg_body)


def _edge_body(src_hbm, dst_hbm, g_hbm, zeros_hbm, out_hbm,
               si, di, rows, acc, sem):
    c = lax.axis_index("c")
    s = lax.axis_index("s")
    wid = c * NS + s

    @pl.when(c == 0)
    def _():
        _stripe(s, lambda r0, nr: pltpu.sync_copy(
            g_hbm.at[pl.ds(r0, nr)], acc.at[pl.ds(r0, nr)]))

    @pl.when(c == 1)
    def _():
        _stripe(s, lambda r0, nr: pltpu.sync_copy(
            zeros_hbm.at[pl.ds(r0, nr)], acc.at[pl.ds(r0, nr)]))

    plsc.subcore_barrier()
    base = wid * (T_CH * CHUNK)

    def step(i, carry):
        off = pl.multiple_of(base + i * CHUNK, CHUNK)
        pltpu.sync_copy(src_hbm.at[pl.ds(off, CHUNK)], si)
        pltpu.sync_copy(dst_hbm.at[pl.ds(off, CHUNK)], di)
        pltpu.async_copy(g_hbm.at[si], rows, sem).wait()
        pltpu.sync_copy(rows, acc.at[di], add=True)
        return carry

    lax.fori_loop(0, T_CH, step, 0)
    plsc.subcore_barrier()
    _stripe(s, lambda r0, nr: pltpu.sync_copy(
        acc.at[pl.ds(r0, nr)], out_hbm.at[c, pl.ds(r0, nr)]))


_edge_call = functools.partial(
    pl.kernel,
    out_type=jax.ShapeDtypeStruct((NC, N, F), jnp.float32),
    mesh=_mesh,
    scratch_types=[
        pltpu.VMEM((CHUNK,), jnp.int32),
        pltpu.VMEM((CHUNK,), jnp.int32),
        pltpu.VMEM((CHUNK, F), jnp.float32),
        pltpu.VMEM_SHARED((N_ACC, F), jnp.float32),
        pltpu.SemaphoreType.DMA,
    ],
)(_edge_body)


BR = 1000


def _dinv_of(cnt_ref):
    return lax.rsqrt(cnt_ref[0][:, 0:1] + 1.0)


def _tc_first_body(x_ref, w_ref, c0_ref, c1_ref, o_ref):
    dinv = lax.rsqrt(c0_ref[0][:, 0:1] + c1_ref[0][:, 0:1] + 1.0)
    h = jnp.dot(x_ref[...], w_ref[...], preferred_element_type=jnp.float32)
    o_ref[...] = h * dinv


def _tc_mid_body(s0_ref, s1_ref, w_ref, b_ref, c0_ref, c1_ref, o_ref):
    dinv = lax.rsqrt(c0_ref[0][:, 0:1] + c1_ref[0][:, 0:1] + 1.0)
    a = jnp.tanh((s0_ref[0] + s1_ref[0]) * dinv + b_ref[...])
    h = jnp.dot(a, w_ref[...], preferred_element_type=jnp.float32)
    o_ref[...] = h * dinv


def _tc_last_body(s0_ref, s1_ref, b_ref, c0_ref, c1_ref, o_ref):
    dinv = lax.rsqrt(c0_ref[0][:, 0:1] + c1_ref[0][:, 0:1] + 1.0)
    o_ref[...] = jnp.tanh((s0_ref[0] + s1_ref[0]) * dinv + b_ref[...])


def _spec_rows():
    return pl.BlockSpec((BR, F), lambda i: (i, 0))


def _spec_plane(p):
    return pl.BlockSpec((1, BR, F), lambda i, p=p: (p, i, 0))


def _spec_cnt(p):
    return pl.BlockSpec((1, BR, DW), lambda i, p=p: (p, i, 0))


def _spec_w():
    return pl.BlockSpec((F, F), lambda i: (0, 0))


def _spec_b():
    return pl.BlockSpec((1, F), lambda i: (0, 0))


_out_nf = jax.ShapeDtypeStruct((N, F), jnp.float32)
_grid = (N // BR,)

_tc_first = pl.pallas_call(
    _tc_first_body, grid=_grid,
    in_specs=[_spec_rows(), _spec_w(), _spec_cnt(0), _spec_cnt(1)],
    out_specs=_spec_rows(), out_shape=_out_nf)

_tc_mid = pl.pallas_call(
    _tc_mid_body, grid=_grid,
    in_specs=[_spec_plane(0), _spec_plane(1), _spec_w(), _spec_b(),
              _spec_cnt(0), _spec_cnt(1)],
    out_specs=_spec_rows(), out_shape=_out_nf)

_tc_last = pl.pallas_call(
    _tc_last_body, grid=_grid,
    in_specs=[_spec_plane(0), _spec_plane(1), _spec_b(),
              _spec_cnt(0), _spec_cnt(1)],
    out_specs=_spec_rows(), out_shape=_out_nf)


@jax.jit
def _run(x, src, dst, W0, b0, W1, b1, W2, b2, W3, b3):
    pad = E_PAD - E
    src_p = jnp.concatenate([src, jnp.zeros((pad,), jnp.int32)])
    dst_p = jnp.concatenate([dst, jnp.full((pad,), N, jnp.int32)])
    zeros_nf = jnp.zeros((N, F), jnp.float32)
    ones_chunk = jnp.ones((CHUNK, DW), jnp.float32)

    cnt = _deg_call(dst_p.reshape(NW, T_CH, CHUNK), zeros_nf, ones_chunk)

    g = _tc_first(x, W0, cnt, cnt)
    s = _edge_call(src_p, dst_p, g, zeros_nf)
    g = _tc_mid(s, s, W1, b0.reshape(1, F), cnt, cnt)
    s = _edge_call(src_p, dst_p, g, zeros_nf)
    g = _tc_mid(s, s, W2, b1.reshape(1, F), cnt, cnt)
    s = _edge_call(src_p, dst_p, g, zeros_nf)
    g = _tc_mid(s, s, W3, b2.reshape(1, F), cnt, cnt)
    s = _edge_call(src_p, dst_p, g, zeros_nf)
    return _tc_last(s, s, b3.reshape(1, F), cnt, cnt)


def kernel(x, edge_index, W0, b0, W1, b1, W2, b2, W3, b3):
    src = edge_index[0].astype(jnp.int32)
    dst = edge_index[1].astype(jnp.int32)
    return _run(x, src, dst, W0, b0, W1, b1, W2, b2, W3, b3)


# speedup vs baseline: 1.6232x; 1.6232x over previous
import functools

import jax
import jax.numpy as jnp
from jax import lax
from jax.experimental import pallas as pl
from jax.experimental.pallas import tpu as pltpu
from jax.experimental.pallas import tpu_sc as plsc

N = 10000
E = 320000
F = 128

NC = 2
NS = 16
NW = NC * NS
CHUNK = 128
T_CH = -(-E // (NW * CHUNK))
E_PAD = NW * CHUNK * T_CH
RPT = 624
RPT_LAST = N - (NS - 1) * RPT
N_ACC = N + 8
DW = F

_mesh = plsc.VectorSubcoreMesh(core_axis_name="c", subcore_axis_name="s")


def _stripe(s, fn):
    @pl.when(s < NS - 1)
    def _():
        fn(pl.multiple_of(s * RPT, 8), RPT)

    @pl.when(s == NS - 1)
    def _():
        fn((NS - 1) * RPT, RPT_LAST)


def _deg_body(dst_hbm, zeros_hbm, ones_hbm, out_hbm, di2d, ones_v, acc):
    c = lax.axis_index("c")
    s = lax.axis_index("s")
    wid = c * NS + s
    _stripe(s, lambda r0, nr: pltpu.sync_copy(
        zeros_hbm.at[pl.ds(r0, nr)], acc.at[pl.ds(r0, nr)]))
    pltpu.sync_copy(ones_hbm, ones_v)
    pltpu.sync_copy(dst_hbm.at[wid], di2d)
    plsc.subcore_barrier()

    def step(i, carry):
        pltpu.sync_copy(ones_v, acc.at[di2d.at[i]], add=True)
        return carry

    lax.fori_loop(0, T_CH, step, 0)
    plsc.subcore_barrier()
    _stripe(s, lambda r0, nr: pltpu.sync_copy(
        acc.at[pl.ds(r0, nr)], out_hbm.at[c, pl.ds(r0, nr)]))


_deg_call = functools.partial(
    pl.kernel,
    out_type=jax.ShapeDtypeStruct((NC, N, DW), jnp.float32),
    mesh=_mesh,
    scratch_types=[
        pltpu.VMEM((T_CH, CHUNK), jnp.int32),
        pltpu.VMEM((CHUNK, DW), jnp.float32),
        pltpu.VMEM_SHARED((N_ACC, DW), jnp.float32),
    ],
)(_deg_body)


def _edge_body(src_hbm, dst_hbm, g_hbm, zeros_hbm, out_hbm,
               si, di, rows, acc, sem):
    c = lax.axis_index("c")
    s = lax.axis_index("s")
    wid = c * NS + s

    @pl.when(c == 0)
    def _():
        _stripe(s, lambda r0, nr: pltpu.sync_copy(
            g_hbm.at[pl.ds(r0, nr)], acc.at[pl.ds(r0, nr)]))

    @pl.when(c == 1)
    def _():
        _stripe(s, lambda r0, nr: pltpu.sync_copy(
            zeros_hbm.at[pl.ds(r0, nr)], acc.at[pl.ds(r0, nr)]))

    plsc.subcore_barrier()
    base = wid * (T_CH * CHUNK)

    def step(i, carry):
        off = pl.multiple_of(base + i * CHUNK, CHUNK)
        pltpu.sync_copy(src_hbm.at[pl.ds(off, CHUNK)], si)
        pltpu.sync_copy(dst_hbm.at[pl.ds(off, CHUNK)], di)
        pltpu.async_copy(g_hbm.at[si], rows, sem).wait()
        pltpu.sync_copy(rows, acc.at[di], add=True)
        return carry

    lax.fori_loop(0, T_CH, step, 0)
    plsc.subcore_barrier()
    _stripe(s, lambda r0, nr: pltpu.sync_copy(
        acc.at[pl.ds(r0, nr)], out_hbm.at[c, pl.ds(r0, nr)]))


_edge_call = functools.partial(
    pl.kernel,
    out_type=jax.ShapeDtypeStruct((NC, N, F), jnp.float32),
    mesh=_mesh,
    scratch_types=[
        pltpu.VMEM((CHUNK,), jnp.int32),
        pltpu.VMEM((CHUNK,), jnp.int32),
        pltpu.VMEM((CHUNK, F), jnp.float32),
        pltpu.VMEM_SHARED((N_ACC, F), jnp.float32),
        pltpu.SemaphoreType.DMA,
    ],
)(_edge_body)


BR = 1000


def _tc_first_body(x_ref, w_ref, c0_ref, c1_ref, o_ref):
    dinv = lax.rsqrt(c0_ref[0][:, 0:1] + c1_ref[0][:, 0:1] + 1.0)
    h = jnp.dot(x_ref[...], w_ref[...], preferred_element_type=jnp.float32)
    o_ref[...] = h * dinv


def _tc_mid_body(s0_ref, s1_ref, w_ref, b_ref, c0_ref, c1_ref, o_ref):
    dinv = lax.rsqrt(c0_ref[0][:, 0:1] + c1_ref[0][:, 0:1] + 1.0)
    a = jnp.tanh((s0_ref[0] + s1_ref[0]) * dinv + b_ref[...])
    h = jnp.dot(a, w_ref[...], preferred_element_type=jnp.float32)
    o_ref[...] = h * dinv


def _tc_last_body(s0_ref, s1_ref, b_ref, c0_ref, c1_ref, o_ref):
    dinv = lax.rsqrt(c0_ref[0][:, 0:1] + c1_ref[0][:, 0:1] + 1.0)
    o_ref[...] = jnp.tanh((s0_ref[0] + s1_ref[0]) * dinv + b_ref[...])


def _spec_rows():
    return pl.BlockSpec((BR, F), lambda i: (i, 0))


def _spec_plane(p):
    return pl.BlockSpec((1, BR, F), lambda i, p=p: (p, i, 0))


def _spec_cnt(p):
    return pl.BlockSpec((1, BR, DW), lambda i, p=p: (p, i, 0))


def _spec_w():
    return pl.BlockSpec((F, F), lambda i: (0, 0))


def _spec_b():
    return pl.BlockSpec((1, F), lambda i: (0, 0))


_out_nf = jax.ShapeDtypeStruct((N, F), jnp.float32)
_grid = (N // BR,)

_tc_first = pl.pallas_call(
    _tc_first_body, grid=_grid,
    in_specs=[_spec_rows(), _spec_w(), _spec_cnt(0), _spec_cnt(1)],
    out_specs=_spec_rows(), out_shape=_out_nf)

_tc_mid = pl.pallas_call(
    _tc_mid_body, grid=_grid,
    in_specs=[_spec_plane(0), _spec_plane(1), _spec_w(), _spec_b(),
              _spec_cnt(0), _spec_cnt(1)],
    out_specs=_spec_rows(), out_shape=_out_nf)

_tc_last = pl.pallas_call(
    _tc_last_body, grid=_grid,
    in_specs=[_spec_plane(0), _spec_plane(1), _spec_b(),
              _spec_cnt(0), _spec_cnt(1)],
    out_specs=_spec_rows(), out_shape=_out_nf)


@jax.jit
def _run(x, src, dst, W0, b0, W1, b1, W2, b2, W3, b3):
    pad = E_PAD - E
    src_p = jnp.concatenate([src, jnp.zeros((pad,), jnp.int32)])
    dst_p = jnp.concatenate([dst, jnp.full((pad,), N, jnp.int32)])
    zeros_nf = jnp.zeros((N, F), jnp.float32)
    ones_chunk = jnp.ones((CHUNK, DW), jnp.float32)

    cnt = _deg_call(dst_p.reshape(NW, T_CH, CHUNK), zeros_nf, ones_chunk)

    g = _tc_first(x, W0, cnt, cnt)
    s = _edge_call(src_p, dst_p, g, zeros_nf)
    g = _tc_mid(s, s, W1, b0.reshape(1, F), cnt, cnt)
    s = _edge_call(src_p, dst_p, g, zeros_nf)
    g = _tc_mid(s, s, W2, b1.reshape(1, F), cnt, cnt)
    s = _edge_call(src_p, dst_p, g, zeros_nf)
    g = _tc_mid(s, s, W3, b2.reshape(1, F), cnt, cnt)
    s = _edge_call(src_p, dst_p, g, zeros_nf)
    return _tc_last(s, s, b3.reshape(1, F), cnt, cnt)


def kernel(x, edge_index, W0, b0, W1, b1, W2, b2, W3, b3):
    src = edge_index[0].astype(jnp.int32)
    dst = edge_index[1].astype(jnp.int32)
    return _run(x, src, dst, W0, b0, W1, b1, W2, b2, W3, b3)
